# baseline (speedup 1.0000x reference)
import functools

import jax
import jax.numpy as jnp
from jax import lax
from jax.experimental import pallas as pl
from jax.experimental.pallas import tpu as pltpu
from jax.experimental.pallas import tpu_sc as plsc

NC = 2
NS = 16
NW = NC * NS

DP = 128


_ROW_BLK = 2048


def _transform_body(pcat_ref, wt_ref, b_ref, out_ref):
    out_ref[...] = (
        lax.dot_general(pcat_ref[...], wt_ref[...],
                        dimension_numbers=(((0,), (0,)), ((), ())),
                        preferred_element_type=jnp.float32,
                        precision=lax.Precision.HIGHEST)
        + b_ref[...]
    )


def _transform_table(pcat, wt, b2d):
    d, v = pcat.shape
    grid = (v + _ROW_BLK - 1) // _ROW_BLK
    return pl.pallas_call(
        _transform_body,
        grid=(grid,),
        in_specs=[
            pl.BlockSpec((d, _ROW_BLK), lambda i: (0, i)),
            pl.BlockSpec((d, DP), lambda i: (0, 0)),
            pl.BlockSpec((1, DP), lambda i: (0, 0)),
        ],
        out_specs=pl.BlockSpec((_ROW_BLK, DP), lambda i: (i, 0)),
        out_shape=jax.ShapeDtypeStruct((v, DP), jnp.float32),
    )(pcat, wt, b2d)


_BPW = 128
_TL = 2


def _make_gather(bsz, seq, d):
    n_groups = seq // _TL
    mesh = plsc.VectorSubcoreMesh(
        core_axis_name="c", subcore_axis_name="s",
        num_cores=NC, num_subcores=NS)

    @functools.partial(
        pl.kernel,
        out_type=jax.ShapeDtypeStruct((seq, d, bsz), jnp.float32),
        mesh=mesh,
        scratch_types=[
            pltpu.VMEM((seq, _BPW), jnp.int32),
            pltpu.VMEM((2, _TL * _BPW, DP), jnp.float32),
            pltpu.VMEM((2, _TL, d, _BPW), jnp.float32),
            pltpu.SemaphoreType.DMA,
            pltpu.SemaphoreType.DMA,
        ],
        compiler_params=pltpu.CompilerParams(
            use_tc_tiling_on_sc=True, needs_layout_passes=False),
    )
    def gather(table_hbm, idxt_hbm, out_hbm, idx_v, gbuf, tbuf, sem0, sem1):
        wid = lax.axis_index("s") * NC + lax.axis_index("c")
        b0 = wid * _BPW
        pltpu.sync_copy(idxt_hbm.at[:, pl.ds(b0, _BPW)], idx_v)
        sems = (sem0, sem1)
        lane = lax.iota(jnp.int32, 16)

        def fire(g, p):
            for c in range(_TL):
                pltpu.async_copy(
                    table_hbm.at[idx_v.at[g * _TL + c]],
                    gbuf.at[p, pl.ds(c * _BPW, _BPW)],
                    sems[p])

        def drain(p):
            for c in range(_TL):
                pltpu.make_async_copy(
                    table_hbm.at[idx_v.at[0]],
                    gbuf.at[p, pl.ds(c * _BPW, _BPW)],
                    sems[p]).wait()

        def transpose(p):
            @plsc.parallel_loop(0, d, 1, unroll=4)
            def tk(k):
                cols = lane * 0 + k
                for c in range(_TL):
                    for g in range(_BPW // 16):
                        rows = lane + (c * _BPW + g * 16)
                        v = plsc.load_gather(gbuf.at[p], [rows, cols])
                        tbuf[p, c, k, pl.ds(g * 16, 16)] = v

        def write(g, p):
            pltpu.sync_copy(
                tbuf.at[p],
                out_hbm.at[pl.ds(g * _TL, _TL), :, pl.ds(b0, _BPW)])

        fire(0, 0)

        def body(i2, carry):
            i = i2 * 2
            fire(i + 1, 1)
            drain(0)
            transpose(0)
            write(i, 0)
            fire(i + 2, 0)
            drain(1)
            transpose(1)
            write(i + 1, 1)
            return carry

        lax.fori_loop(0, n_groups // 2 - 1, body, 0)
        fire(n_groups - 1, 1)
        drain(0)
        transpose(0)
        write(n_groups - 2, 0)
        drain(1)
        transpose(1)
        write(n_groups - 1, 1)

    return gather


def kernel(nodes, pca_matrix, W, b):
    bsz, seq = nodes.shape
    d = pca_matrix.shape[1]

    wt = jnp.zeros((d, DP), jnp.float32).at[:, :d].set(W.T)
    b2d = jnp.zeros((1, DP), jnp.float32).at[:, :d].set(b)
    table = _transform_table(pca_matrix.T, wt, b2d)

    idxt = nodes.T.astype(jnp.int32)
    out_t = _make_gather(bsz, seq, d)(table, idxt)
    return out_t.transpose(2, 0, 1)


# speedup vs baseline: 1.4156x; 1.4156x over previous
import functools

import jax
import jax.numpy as jnp
from jax import lax
from jax.experimental import pallas as pl
from jax.experimental.pallas import tpu as pltpu
from jax.experimental.pallas import tpu_sc as plsc

NC = 2
NS = 16
NW = NC * NS

DP = 128


_ROW_BLK = 2048


def _transform_body(pcat_ref, wt_ref, b_ref, out_ref):
    out_ref[...] = (
        lax.dot_general(pcat_ref[...], wt_ref[...],
                        dimension_numbers=(((0,), (0,)), ((), ())),
                        preferred_element_type=jnp.float32,
                        precision=lax.Precision.HIGHEST)
        + b_ref[...]
    )


def _transform_table(pcat, wt, b2d):
    d, v = pcat.shape
    grid = (v + _ROW_BLK - 1) // _ROW_BLK
    return pl.pallas_call(
        _transform_body,
        grid=(grid,),
        in_specs=[
            pl.BlockSpec((d, _ROW_BLK), lambda i: (0, i)),
            pl.BlockSpec((d, DP), lambda i: (0, 0)),
            pl.BlockSpec((1, DP), lambda i: (0, 0)),
        ],
        out_specs=pl.BlockSpec((_ROW_BLK, DP), lambda i: (i, 0)),
        out_shape=jax.ShapeDtypeStruct((v, DP), jnp.float32),
    )(pcat, wt, b2d)


_CHUNK = 128
_K = 2


def _make_gather(n_flat):
    per_w = n_flat // NW
    n_chunks = per_w // _CHUNK
    n_groups = n_chunks // _K
    grp = _K * _CHUNK
    mesh = plsc.VectorSubcoreMesh(
        core_axis_name="c", subcore_axis_name="s",
        num_cores=NC, num_subcores=NS)

    @functools.partial(
        pl.kernel,
        out_type=jax.ShapeDtypeStruct((n_flat, DP), jnp.float32),
        mesh=mesh,
        scratch_types=[
            pltpu.VMEM((n_chunks, _CHUNK), jnp.int32),
            pltpu.VMEM((2, grp, DP), jnp.float32),
            pltpu.SemaphoreType.DMA,
            pltpu.SemaphoreType.DMA,
        ],
        compiler_params=pltpu.CompilerParams(use_tc_tiling_on_sc=True),
    )
    def gather(table_hbm, idx_hbm, out_hbm, idx_v, rows_v, sem0, sem1):
        wid = lax.axis_index("s") * NC + lax.axis_index("c")
        pltpu.sync_copy(idx_hbm.at[wid], idx_v)
        base = wid * per_w
        sems = (sem0, sem1)

        def fire(g, b):
            for k in range(_K):
                pltpu.async_copy(
                    table_hbm.at[idx_v.at[g * _K + k]],
                    rows_v.at[b, pl.ds(k * _CHUNK, _CHUNK)],
                    sems[b])

        def drain(b):
            for k in range(_K):
                pltpu.make_async_copy(
                    table_hbm.at[idx_v.at[0]],
                    rows_v.at[b, pl.ds(k * _CHUNK, _CHUNK)],
                    sems[b]).wait()

        def write(g, b):
            pltpu.sync_copy(rows_v.at[b],
                            out_hbm.at[pl.ds(base + g * grp, grp)])

        fire(0, 0)

        def body(i2, carry):
            i = i2 * 2
            fire(i + 1, 1)
            drain(0)
            write(i, 0)
            fire(i + 2, 0)
            drain(1)
            write(i + 1, 1)
            return carry

        lax.fori_loop(0, n_groups // 2 - 1, body, 0)
        i = n_groups - 2
        fire(i + 1, 1)
        drain(0)
        write(i, 0)
        drain(1)
        write(i + 1, 1)

    return gather


def kernel(nodes, pca_matrix, W, b):
    bsz, seq = nodes.shape
    d = pca_matrix.shape[1]
    n_flat = bsz * seq

    wt = jnp.zeros((d, DP), jnp.float32).at[:, :d].set(W.T)
    b2d = jnp.zeros((1, DP), jnp.float32).at[:, :d].set(b)
    table = _transform_table(pca_matrix.T, wt, b2d)

    per_w = n_flat // NW
    idx = nodes.reshape(-1).astype(jnp.int32).reshape(NW, per_w // _CHUNK, _CHUNK)
    out = _make_gather(n_flat)(table, idx)
    return out[:, :d].reshape(bsz, seq, d)


# speedup vs baseline: 1.4539x; 1.0271x over previous
import functools

import jax
import jax.numpy as jnp
from jax import lax
from jax.experimental import pallas as pl
from jax.experimental.pallas import tpu as pltpu
from jax.experimental.pallas import tpu_sc as plsc

NC = 2
NS = 16
NW = NC * NS

DP = 128


_ROW_BLK = 8192


def _transform_body(pcat_ref, wt_ref, b_ref, out_ref):
    out_ref[...] = (
        lax.dot_general(pcat_ref[...], wt_ref[...],
                        dimension_numbers=(((0,), (0,)), ((), ())),
                        preferred_element_type=jnp.float32,
                        precision=lax.Precision.HIGHEST)
        + b_ref[...]
    )


def _transform_table(pcat, wt, b2d):
    d, v = pcat.shape
    grid = (v + _ROW_BLK - 1) // _ROW_BLK
    return pl.pallas_call(
        _transform_body,
        grid=(grid,),
        in_specs=[
            pl.BlockSpec((d, _ROW_BLK), lambda i: (0, i)),
            pl.BlockSpec((d, DP), lambda i: (0, 0)),
            pl.BlockSpec((1, DP), lambda i: (0, 0)),
        ],
        out_specs=pl.BlockSpec((_ROW_BLK, DP), lambda i: (i, 0)),
        out_shape=jax.ShapeDtypeStruct((v, DP), jnp.float32),
    )(pcat, wt, b2d)


_CHUNK = 128
_K = 2


def _make_gather(n_flat):
    per_w = n_flat // NW
    n_chunks = per_w // _CHUNK
    n_groups = n_chunks // _K
    grp = _K * _CHUNK
    mesh = plsc.VectorSubcoreMesh(
        core_axis_name="c", subcore_axis_name="s",
        num_cores=NC, num_subcores=NS)

    @functools.partial(
        pl.kernel,
        out_type=jax.ShapeDtypeStruct((n_flat, DP), jnp.float32),
        mesh=mesh,
        scratch_types=[
            pltpu.VMEM((n_chunks, _CHUNK), jnp.int32),
            pltpu.VMEM((2, grp, DP), jnp.float32),
            pltpu.SemaphoreType.DMA,
            pltpu.SemaphoreType.DMA,
        ],
        compiler_params=pltpu.CompilerParams(use_tc_tiling_on_sc=True),
    )
    def gather(table_hbm, idx_hbm, out_hbm, idx_v, rows_v, sem0, sem1):
        wid = lax.axis_index("s") * NC + lax.axis_index("c")
        pltpu.sync_copy(idx_hbm.at[wid], idx_v)
        base = wid * per_w
        sems = (sem0, sem1)

        def fire(g, b):
            for k in range(_K):
                pltpu.async_copy(
                    table_hbm.at[idx_v.at[g * _K + k]],
                    rows_v.at[b, pl.ds(k * _CHUNK, _CHUNK)],
                    sems[b])

        def drain(b):
            for k in range(_K):
                pltpu.make_async_copy(
                    table_hbm.at[idx_v.at[0]],
                    rows_v.at[b, pl.ds(k * _CHUNK, _CHUNK)],
                    sems[b]).wait()

        def write(g, b):
            pltpu.sync_copy(rows_v.at[b],
                            out_hbm.at[pl.ds(base + g * grp, grp)])

        fire(0, 0)

        def body(i2, carry):
            i = i2 * 2
            fire(i + 1, 1)
            drain(0)
            write(i, 0)
            fire(i + 2, 0)
            drain(1)
            write(i + 1, 1)
            return carry

        lax.fori_loop(0, n_groups // 2 - 1, body, 0)
        i = n_groups - 2
        fire(i + 1, 1)
        drain(0)
        write(i, 0)
        drain(1)
        write(i + 1, 1)

    return gather


def kernel(nodes, pca_matrix, W, b):
    bsz, seq = nodes.shape
    d = pca_matrix.shape[1]
    n_flat = bsz * seq

    wt = jnp.zeros((d, DP), jnp.float32).at[:, :d].set(W.T)
    b2d = jnp.zeros((1, DP), jnp.float32).at[:, :d].set(b)
    table = _transform_table(pca_matrix.T, wt, b2d)

    per_w = n_flat // NW
    idx = nodes.reshape(-1).astype(jnp.int32).reshape(NW, per_w // _CHUNK, _CHUNK)
    out = _make_gather(n_flat)(table, idx)
    return out[:, :d].reshape(bsz, seq, d)
